# baseline (speedup 1.0000x reference)
import jax
import jax.numpy as jnp
from jax import lax
from jax.experimental import pallas as pl
from jax.experimental.pallas import tpu as pltpu
from jax.experimental.pallas import tpu_sc as plsc

B = 4096
N_ITEMS = 200
D = 64
T = 50
SLAB = N_ITEMS * D
NC = 2
NS = 16
NW = NC * NS
BPW = B // NW
GRP = 4
IDG = (T + 15) // 16


def _body(table, ids, out, idx_v, sa, sb, oa, ob, gs0, gs1, os0, os1):
    wid = lax.axis_index("s") * NC + lax.axis_index("c")
    b0 = wid * BPW

    slabs = [sa, sb]
    outgs = [oa, ob]
    gsems = [gs0, gs1]
    osems = [os0, os1]

    pltpu.sync_copy(ids.at[pl.ds(b0, BPW)], idx_v)

    def fetch_slab(b, buf, sem):
        pltpu.async_copy(table.at[pl.ds(b * SLAB, SLAB)], buf, sem)

    fetch_slab(b0, sa, gs0)

    def select_rows(k, slab_b, outg_b, slot):
        def g_body(g, carry):
            o = lax.min(g * 16, T - 16)
            tv = idx_v[k, pl.ds(o, 16)]
            for i in range(16):
                row = tv[i] * D
                for q in range(D // 16):
                    outg_b[slot, o + i, pl.ds(q * 16, 16)] = (
                        slab_b[pl.ds(row + q * 16, 16)])
            return carry

        lax.fori_loop(0, IDG, g_body, 0)

    def g8_body(g8, carry):
        for j in range(8):
            k = g8 * 8 + j
            cur = j % 2
            obuf = j // 4

            @pl.when(k + 1 < BPW)
            def _():
                fetch_slab(b0 + k + 1, slabs[1 - cur], gsems[1 - cur])

            if j % 4 == 0:
                @pl.when(k >= 2 * GRP)
                def _():
                    pltpu.make_async_copy(outgs[obuf],
                                          out.at[pl.ds(0, GRP)],
                                          osems[obuf]).wait()

            pltpu.make_async_copy(table.at[pl.ds(0, SLAB)], slabs[cur],
                                  gsems[cur]).wait()
            select_rows(k, slabs[cur], outgs[obuf], j % 4)

            if j % 4 == 3:
                grp0 = k - 3
                pltpu.async_copy(outgs[obuf],
                                 out.at[pl.ds(b0 + grp0, GRP)],
                                 osems[obuf])
        return carry

    lax.fori_loop(0, BPW // 8, g8_body, 0)

    pltpu.make_async_copy(oa, out.at[pl.ds(0, GRP)], os0).wait()
    pltpu.make_async_copy(ob, out.at[pl.ds(0, GRP)], os1).wait()


def kernel(all_embeddings, target_ids):
    ids = target_ids.astype(jnp.int32)
    table = all_embeddings.reshape(B * N_ITEMS * D)
    mesh = plsc.VectorSubcoreMesh(core_axis_name="c", subcore_axis_name="s")
    run = pl.kernel(
        _body,
        mesh=mesh,
        out_type=jax.ShapeDtypeStruct((B, T, D), jnp.float32),
        scratch_types=[
            pltpu.VMEM((BPW, T), jnp.int32),
            pltpu.VMEM((SLAB,), jnp.float32),
            pltpu.VMEM((SLAB,), jnp.float32),
            pltpu.VMEM((GRP, T, D), jnp.float32),
            pltpu.VMEM((GRP, T, D), jnp.float32),
            pltpu.SemaphoreType.DMA,
            pltpu.SemaphoreType.DMA,
            pltpu.SemaphoreType.DMA,
            pltpu.SemaphoreType.DMA,
        ],
    )
    return run(table, ids)


# speedup vs baseline: 1.6144x; 1.6144x over previous
import jax
import jax.numpy as jnp
from jax import lax
from jax.experimental import pallas as pl
from jax.experimental.pallas import tpu as pltpu
from jax.experimental.pallas import tpu_sc as plsc

B = 4096
N_ITEMS = 200
D = 64
T = 50
SR = N_ITEMS // 2
SW = 2 * D
NC = 2
NS = 16
NW = NC * NS
BPW = B // NW
GRP = 4
IDG = (T + 15) // 16


def _body(table, ids, out, idx_v, sa, sb, oa, ob, gs0, gs1, os0, os1):
    wid = lax.axis_index("s") * NC + lax.axis_index("c")
    b0 = wid * BPW

    slabs = [sa, sb]
    outgs = [oa, ob]
    gsems = [gs0, gs1]
    osems = [os0, os1]

    pltpu.sync_copy(ids.at[pl.ds(b0, BPW)], idx_v)

    def fetch_slab(b, buf, sem):
        pltpu.async_copy(table.at[b], buf, sem)

    fetch_slab(b0, sa, gs0)

    def select_rows(k, slab_b, outg_b, slot):
        def g_body(g, carry):
            o = lax.min(g * 16, T - 16)
            tv = idx_v[k, pl.ds(o, 16)]
            for i in range(16):
                sid = tv[i]
                mr = sid >> 1
                mc = (sid & 1) * D
                for q in range(D // 16):
                    outg_b[slot, o + i, pl.ds(q * 16, 16)] = (
                        slab_b[mr, pl.ds(mc + q * 16, 16)])
            return carry

        lax.fori_loop(0, IDG, g_body, 0)

    def g8_body(g8, carry):
        for j in range(8):
            k = g8 * 8 + j
            cur = j % 2
            obuf = j // 4

            @pl.when(k + 1 < BPW)
            def _():
                fetch_slab(b0 + k + 1, slabs[1 - cur], gsems[1 - cur])

            if j % 4 == 0:
                @pl.when(k >= 2 * GRP)
                def _():
                    pltpu.make_async_copy(outgs[obuf],
                                          out.at[pl.ds(0, GRP)],
                                          osems[obuf]).wait()

            pltpu.make_async_copy(table.at[b0], slabs[cur],
                                  gsems[cur]).wait()
            select_rows(k, slabs[cur], outgs[obuf], j % 4)

            if j % 4 == 3:
                grp0 = k - 3
                pltpu.async_copy(outgs[obuf],
                                 out.at[pl.ds(b0 + grp0, GRP)],
                                 osems[obuf])
        return carry

    lax.fori_loop(0, BPW // 8, g8_body, 0)

    pltpu.make_async_copy(oa, out.at[pl.ds(0, GRP)], os0).wait()
    pltpu.make_async_copy(ob, out.at[pl.ds(0, GRP)], os1).wait()


def kernel(all_embeddings, target_ids):
    ids = target_ids.astype(jnp.int32)
    table = all_embeddings.reshape(B, SR, SW)
    mesh = plsc.VectorSubcoreMesh(core_axis_name="c", subcore_axis_name="s")
    run = pl.kernel(
        _body,
        mesh=mesh,
        out_type=jax.ShapeDtypeStruct((B, T, D), jnp.float32),
        scratch_types=[
            pltpu.VMEM((BPW, T), jnp.int32),
            pltpu.VMEM((SR, SW), jnp.float32),
            pltpu.VMEM((SR, SW), jnp.float32),
            pltpu.VMEM((GRP, T, D), jnp.float32),
            pltpu.VMEM((GRP, T, D), jnp.float32),
            pltpu.SemaphoreType.DMA,
            pltpu.SemaphoreType.DMA,
            pltpu.SemaphoreType.DMA,
            pltpu.SemaphoreType.DMA,
        ],
    )
    return run(table, ids)
